# baseline (speedup 1.0000x reference)
import jax
import jax.numpy as jnp
from jax import lax
from jax.experimental import pallas as pl
from jax.experimental.pallas import tpu as pltpu
from jax.experimental.pallas import tpu_sc as plsc

_EMB = 64
_D2 = 2 * _EMB
_B = 16384
_NC, _NS, _L = 2, 16, 16
_NW = _NC * _NS
_BPW = _B // _NW
_CH = 128
_NCHUNK = _BPW // _CH


def _sc_body(head_hbm, rel_hbm, tail_hbm, ent_hbm, relemb_hbm, out_hbm,
             idx_h, idx_r, idx_t, h_rows, r_rows, t_rows, out_v, sem):
    wid = lax.axis_index("s") * _NC + lax.axis_index("c")
    base = wid * _BPW
    iota = lax.iota(jnp.int32, 16)
    for ch in range(_NCHUNK):
        off = base + ch * _CH
        pltpu.sync_copy(head_hbm.at[pl.ds(off, _CH)], idx_h)
        pltpu.sync_copy(rel_hbm.at[pl.ds(off, _CH)], idx_r)
        pltpu.sync_copy(tail_hbm.at[pl.ds(off, _CH)], idx_t)
        c1 = pltpu.async_copy(ent_hbm.at[idx_h], h_rows, sem)
        c2 = pltpu.async_copy(relemb_hbm.at[idx_r], r_rows, sem)
        c3 = pltpu.async_copy(ent_hbm.at[idx_t], t_rows, sem)
        c1.wait()
        c2.wait()
        c3.wait()
        for g in range(_CH // 16):
            row_ids = iota + (g * 16)

            def dbody(d, acc):
                cr = jnp.full((16,), d, jnp.int32)
                ci = cr + _EMB
                hr = plsc.load_gather(h_rows, [row_ids, cr])
                hi = plsc.load_gather(h_rows, [row_ids, ci])
                rr = plsc.load_gather(r_rows, [row_ids, cr])
                ri = plsc.load_gather(r_rows, [row_ids, ci])
                tr = plsc.load_gather(t_rows, [row_ids, cr])
                ti = plsc.load_gather(t_rows, [row_ids, ci])
                a = hr * rr + hi * ri
                b = hr * ri - hi * rr
                return acc + (a * tr - b * ti)

            acc = lax.fori_loop(0, _EMB, dbody, jnp.zeros((16,), jnp.float32))
            out_v[pl.ds(ch * _CH + g * 16, 16)] = acc
    pltpu.sync_copy(out_v, out_hbm.at[pl.ds(base, _BPW)])


@jax.jit
def kernel(head_ids, relation_ids, tail_ids, entity_emb, relation_emb):
    k = pl.kernel(
        _sc_body,
        out_type=jax.ShapeDtypeStruct((_B,), jnp.float32),
        mesh=plsc.VectorSubcoreMesh(core_axis_name="c", subcore_axis_name="s"),
        compiler_params=pltpu.CompilerParams(needs_layout_passes=False),
        scratch_types=[
            pltpu.VMEM((_CH,), jnp.int32),
            pltpu.VMEM((_CH,), jnp.int32),
            pltpu.VMEM((_CH,), jnp.int32),
            pltpu.VMEM((_CH, _D2), jnp.float32),
            pltpu.VMEM((_CH, _D2), jnp.float32),
            pltpu.VMEM((_CH, _D2), jnp.float32),
            pltpu.VMEM((_BPW,), jnp.float32),
            pltpu.SemaphoreType.DMA,
        ],
    )
    return k(head_ids, relation_ids, tail_ids, entity_emb, relation_emb)


# speedup vs baseline: 1.0716x; 1.0716x over previous
import jax
import jax.numpy as jnp
from jax import lax
from jax.experimental import pallas as pl
from jax.experimental.pallas import tpu as pltpu
from jax.experimental.pallas import tpu_sc as plsc

_EMB = 64
_D2 = 2 * _EMB
_B = 16384
_NC, _NS, _L = 2, 16, 16
_NW = _NC * _NS
_BPW = _B // _NW
_CH = 128
_NCHUNK = _BPW // _CH
_NG = _CH // 16


def _sc_body(head_hbm, rel_hbm, tail_hbm, ent_hbm, relemb_hbm, out_hbm,
             idx_h, idx_r, idx_t, h_rows, r_rows, t_rows, out_v, sems):
    wid = lax.axis_index("s") * _NC + lax.axis_index("c")
    base = wid * _BPW
    iota = lax.iota(jnp.int32, 16)
    pltpu.sync_copy(head_hbm.at[pl.ds(base, _BPW)], idx_h)
    pltpu.sync_copy(rel_hbm.at[pl.ds(base, _BPW)], idx_r)
    pltpu.sync_copy(tail_hbm.at[pl.ds(base, _BPW)], idx_t)

    def start(ch):
        slot = ch & 1
        sl = pl.ds(ch * _CH, _CH)
        return (
            pltpu.async_copy(ent_hbm.at[idx_h.at[sl]], h_rows.at[slot], sems.at[slot]),
            pltpu.async_copy(relemb_hbm.at[idx_r.at[sl]], r_rows.at[slot], sems.at[slot]),
            pltpu.async_copy(ent_hbm.at[idx_t.at[sl]], t_rows.at[slot], sems.at[slot]),
        )

    row_ids = [iota + g * 16 for g in range(_NG)]
    pending = start(0)
    for ch in range(_NCHUNK):
        slot = ch & 1
        for c in pending:
            c.wait()
        if ch + 1 < _NCHUNK:
            pending = start(ch + 1)

        def dbody(d, accs):
            cr = jnp.full((16,), d, jnp.int32)
            ci = cr + _EMB
            out = []
            for g in range(_NG):
                rid = row_ids[g]
                hr = plsc.load_gather(h_rows.at[slot], [rid, cr])
                hi = plsc.load_gather(h_rows.at[slot], [rid, ci])
                rr = plsc.load_gather(r_rows.at[slot], [rid, cr])
                ri = plsc.load_gather(r_rows.at[slot], [rid, ci])
                tr = plsc.load_gather(t_rows.at[slot], [rid, cr])
                ti = plsc.load_gather(t_rows.at[slot], [rid, ci])
                a = hr * rr + hi * ri
                b = hr * ri - hi * rr
                out.append(accs[g] + (a * tr - b * ti))
            return tuple(out)

        zero = jnp.zeros((16,), jnp.float32)
        accs = lax.fori_loop(0, _EMB, dbody, (zero,) * _NG)
        for g in range(_NG):
            out_v[pl.ds(ch * _CH + g * 16, 16)] = accs[g]
    pltpu.sync_copy(out_v, out_hbm.at[pl.ds(base, _BPW)])


@jax.jit
def kernel(head_ids, relation_ids, tail_ids, entity_emb, relation_emb):
    k = pl.kernel(
        _sc_body,
        out_type=jax.ShapeDtypeStruct((_B,), jnp.float32),
        mesh=plsc.VectorSubcoreMesh(core_axis_name="c", subcore_axis_name="s"),
        compiler_params=pltpu.CompilerParams(needs_layout_passes=False),
        scratch_types=[
            pltpu.VMEM((_BPW,), jnp.int32),
            pltpu.VMEM((_BPW,), jnp.int32),
            pltpu.VMEM((_BPW,), jnp.int32),
            pltpu.VMEM((2, _CH, _D2), jnp.float32),
            pltpu.VMEM((2, _CH, _D2), jnp.float32),
            pltpu.VMEM((2, _CH, _D2), jnp.float32),
            pltpu.VMEM((_BPW,), jnp.float32),
            pltpu.SemaphoreType.DMA((2,)),
        ],
    )
    return k(head_ids, relation_ids, tail_ids, entity_emb, relation_emb)


# speedup vs baseline: 3.9451x; 3.6817x over previous
import jax
import jax.numpy as jnp
from jax import lax
from jax.experimental import pallas as pl
from jax.experimental.pallas import tpu as pltpu
from jax.experimental.pallas import tpu_sc as plsc

_EMB = 64
_D2 = 2 * _EMB
_B = 16384
_NC, _NS, _L = 2, 16, 16
_NW = _NC * _NS
_BPW = _B // _NW
_CH = 128
_NCHUNK = _BPW // _CH
_NG = _CH // 16


def _sc_body(head_hbm, rel_hbm, tail_hbm, ent_hbm, relemb_hbm, out_hbm,
             idx_h, idx_r, idx_t, h_rows, r_rows, t_rows, out_v, sems):
    wid = lax.axis_index("s") * _NC + lax.axis_index("c")
    base = wid * _BPW
    iota = lax.iota(jnp.int32, 16)
    pltpu.sync_copy(head_hbm.at[pl.ds(base, _BPW)], idx_h)
    pltpu.sync_copy(rel_hbm.at[pl.ds(base, _BPW)], idx_r)
    pltpu.sync_copy(tail_hbm.at[pl.ds(base, _BPW)], idx_t)

    def start(ch):
        slot = ch & 1
        sl = pl.ds(ch * _CH, _CH)
        return (
            pltpu.async_copy(ent_hbm.at[idx_h.at[sl]], h_rows.at[slot], sems.at[slot]),
            pltpu.async_copy(relemb_hbm.at[idx_r.at[sl]], r_rows.at[slot], sems.at[slot]),
            pltpu.async_copy(ent_hbm.at[idx_t.at[sl]], t_rows.at[slot], sems.at[slot]),
        )

    lane15 = iota == 15
    pending = start(0)
    for ch in range(_NCHUNK):
        slot = ch & 1
        for c in pending:
            c.wait()
        if ch + 1 < _NCHUNK:
            pending = start(ch + 1)

        @plsc.parallel_loop(0, _CH, unroll=4)
        def _rows(i):
            partial = jnp.zeros((16,), jnp.float32)
            for c in range(_EMB // 16):
                re = pl.ds(c * 16, 16)
                im = pl.ds(_EMB + c * 16, 16)
                hr = h_rows[slot, i, re]
                hi = h_rows[slot, i, im]
                rr = r_rows[slot, i, re]
                ri = r_rows[slot, i, im]
                tr = t_rows[slot, i, re]
                ti = t_rows[slot, i, im]
                a = hr * rr + hi * ri
                b = hr * ri - hi * rr
                partial = partial + (a * tr - b * ti)
            total = plsc.cumsum(partial)
            pos = jnp.full((16,), ch * _CH + i, jnp.int32)
            plsc.store_scatter(out_v, [pos], total, mask=lane15)
    pltpu.sync_copy(out_v, out_hbm.at[pl.ds(base, _BPW)])


@jax.jit
def kernel(head_ids, relation_ids, tail_ids, entity_emb, relation_emb):
    k = pl.kernel(
        _sc_body,
        out_type=jax.ShapeDtypeStruct((_B,), jnp.float32),
        mesh=plsc.VectorSubcoreMesh(core_axis_name="c", subcore_axis_name="s"),
        compiler_params=pltpu.CompilerParams(needs_layout_passes=False),
        scratch_types=[
            pltpu.VMEM((_BPW,), jnp.int32),
            pltpu.VMEM((_BPW,), jnp.int32),
            pltpu.VMEM((_BPW,), jnp.int32),
            pltpu.VMEM((2, _CH, _D2), jnp.float32),
            pltpu.VMEM((2, _CH, _D2), jnp.float32),
            pltpu.VMEM((2, _CH, _D2), jnp.float32),
            pltpu.VMEM((_BPW,), jnp.float32),
            pltpu.SemaphoreType.DMA((2,)),
        ],
    )
    return k(head_ids, relation_ids, tail_ids, entity_emb, relation_emb)
